# Initial kernel scaffold; baseline (speedup 1.0000x reference)
import jax
import jax.numpy as jnp
from jax.experimental import pallas as pl


def kernel(x, edge_index, edge_attr, Wl0, Wr0, b0, Wl1, Wr1, b1, Wl2, Wr2, b2, PW1, Pb1, PW2, Pb2, PW3, Pb3):
    raise NotImplementedError("write your pallas kernel here")


# speedup vs baseline: 4.5662x; 4.5662x over previous
import jax
import jax.numpy as jnp
from jax import lax
from jax.experimental import pallas as pl
from jax.experimental.pallas import tpu as pltpu
from jax.experimental.pallas import tpu_sc as plsc

NN = 10000
NE = 320000
DIN = 128
H = 64
W128 = 128
NC = 2
NS = 16
NW = NC * NS
CHE = 128
NCHUNK = NE // CHE
_F32 = jnp.float32

_MESH = plsc.VectorSubcoreMesh(core_axis_name="c", subcore_axis_name="s")

_RPW = 624
_TAIL = NN - NS * _RPW


def _worker_chunks():
    c = lax.axis_index("c")
    s = lax.axis_index("s")
    w = s * NC + c
    full = NCHUNK // NW
    nch = jnp.where(w < NCHUNK - full * NW, full + 1, full)
    return c, s, w, nch


def _sc_agg_body(tab, src2, dst2, z2d,
                 agg_out,
                 idx_s, idx_d, rows, sem, acc):
    c, s, w, nch = _worker_chunks()

    @pl.when(s == 0)
    def _():
        pltpu.sync_copy(z2d, acc)

    plsc.subcore_barrier()

    def body(k, carry):
        ch = w + k * NW
        pltpu.sync_copy(src2.at[ch], idx_s)
        pltpu.sync_copy(dst2.at[ch], idx_d)
        pltpu.async_copy(tab.at[idx_s], rows, sem).wait()
        pltpu.sync_copy(rows, acc.at[idx_d], add=True)
        return carry

    lax.fori_loop(0, nch, body, 0)
    plsc.subcore_barrier()

    pltpu.sync_copy(acc.at[pl.ds(s * _RPW, _RPW)],
                    agg_out.at[c, pl.ds(s * _RPW, _RPW)])

    @pl.when(s == 0)
    def _():
        pltpu.sync_copy(acc.at[pl.ds(NS * _RPW, _TAIL)],
                        agg_out.at[c, pl.ds(NS * _RPW, _TAIL)])


def _sc_gather_body(tab, src2, dst2,
                    gs_out, gd_out,
                    idx_s, idx_d, rows_s, rows_d, sem_s, sem_d):
    c, s, w, nch = _worker_chunks()

    def body(k, carry):
        ch = w + k * NW
        base = ch * CHE
        pltpu.sync_copy(src2.at[ch], idx_s)
        pltpu.sync_copy(dst2.at[ch], idx_d)
        cp_s = pltpu.async_copy(tab.at[idx_s], rows_s, sem_s)
        cp_d = pltpu.async_copy(tab.at[idx_d], rows_d, sem_d)
        cp_s.wait()
        pltpu.sync_copy(rows_s, gs_out.at[pl.ds(base, CHE)])
        cp_d.wait()
        pltpu.sync_copy(rows_d, gd_out.at[pl.ds(base, CHE)])
        return carry

    lax.fori_loop(0, nch, body, 0)


_sc_agg = pl.kernel(
    _sc_agg_body,
    out_type=jax.ShapeDtypeStruct((NC, NN, W128), _F32),
    mesh=_MESH,
    scratch_types=[
        pltpu.VMEM((CHE,), jnp.int32),
        pltpu.VMEM((CHE,), jnp.int32),
        pltpu.VMEM((CHE, W128), _F32),
        pltpu.SemaphoreType.DMA,
        pltpu.VMEM_SHARED((NN, W128), _F32),
    ],
)

_sc_gather = pl.kernel(
    _sc_gather_body,
    out_type=(jax.ShapeDtypeStruct((NE, W128), _F32),
              jax.ShapeDtypeStruct((NE, W128), _F32)),
    mesh=_MESH,
    scratch_types=[
        pltpu.VMEM((CHE,), jnp.int32),
        pltpu.VMEM((CHE,), jnp.int32),
        pltpu.VMEM((CHE, W128), _F32),
        pltpu.VMEM((CHE, W128), _F32),
        pltpu.SemaphoreType.DMA,
        pltpu.SemaphoreType.DMA,
    ],
)


def _mm0_body(x_ref, wpad_ref, e64_ref, wr_ref, br_ref, tab_ref, sf_ref):
    x = x_ref[...]
    tab_ref[...] = jnp.dot(x, wpad_ref[...],
                           preferred_element_type=_F32) + e64_ref[...]
    sf_ref[...] = jnp.dot(x, wr_ref[...],
                          preferred_element_type=_F32) + br_ref[...]


_mm0 = pl.pallas_call(
    _mm0_body,
    out_shape=(jax.ShapeDtypeStruct((NN, W128), _F32),
               jax.ShapeDtypeStruct((NN, H), _F32)),
)


def _combine_first_body(p_ref, sf_ref, wpad_ref, wr_ref, br_ref,
                        tab_ref, sfo_ref, inv_ref):
    agg = p_ref[0] + p_ref[1]
    inv = 1.0 / jnp.maximum(agg[:, H:H + 1], 1.0)
    h = jnp.maximum(agg[:, :H] * inv + sf_ref[...], 0.0)
    tab_ref[...] = jnp.dot(h, wpad_ref[...], preferred_element_type=_F32)
    sfo_ref[...] = jnp.dot(h, wr_ref[...],
                           preferred_element_type=_F32) + br_ref[...]
    inv_ref[...] = inv


_combine_first = pl.pallas_call(
    _combine_first_body,
    out_shape=(jax.ShapeDtypeStruct((NN, W128), _F32),
               jax.ShapeDtypeStruct((NN, H), _F32),
               jax.ShapeDtypeStruct((NN, 1), _F32)),
)


def _combine_mid_body(p_ref, inv_ref, sf_ref, wpad_ref, wr_ref, br_ref,
                      tab_ref, sfo_ref):
    agg = p_ref[0] + p_ref[1]
    h = jnp.maximum(agg[:, :H] * inv_ref[...] + sf_ref[...], 0.0)
    tab_ref[...] = jnp.dot(h, wpad_ref[...], preferred_element_type=_F32)
    sfo_ref[...] = jnp.dot(h, wr_ref[...],
                           preferred_element_type=_F32) + br_ref[...]


_combine_mid = pl.pallas_call(
    _combine_mid_body,
    out_shape=(jax.ShapeDtypeStruct((NN, W128), _F32),
               jax.ShapeDtypeStruct((NN, H), _F32)),
)


def _combine_last_body(p_ref, inv_ref, sf_ref, wc_ref, tab_ref):
    agg = p_ref[0] + p_ref[1]
    h3 = agg[:, :H] * inv_ref[...] + sf_ref[...]
    tab_ref[...] = jnp.dot(h3, wc_ref[...], preferred_element_type=_F32)


_combine_last = pl.pallas_call(
    _combine_last_body,
    out_shape=jax.ShapeDtypeStruct((NN, W128), _F32),
)

_BE = 8000


def _edge_mlp_body(gs, gd, ea, w1, b1, w2, b2, w3, b3, out):
    z1 = gs[:, :H] + gd[:, H:] + jnp.dot(ea[...], w1[...],
                                         preferred_element_type=_F32) + b1[...]
    z1 = jnp.maximum(z1, 0.0)
    z2 = jnp.maximum(jnp.dot(z1, w2[...],
                             preferred_element_type=_F32) + b2[...], 0.0)
    o = jnp.dot(z2, w3[...], preferred_element_type=_F32) + b3[...]
    out[...] = 1.0 / (1.0 + jnp.exp(-o))


_edge_mlp = pl.pallas_call(
    _edge_mlp_body,
    grid=(NE // _BE,),
    in_specs=[
        pl.BlockSpec((_BE, W128), lambda i: (i, 0)),
        pl.BlockSpec((_BE, W128), lambda i: (i, 0)),
        pl.BlockSpec((_BE, 16), lambda i: (i, 0)),
        pl.BlockSpec((16, H), lambda i: (0, 0)),
        pl.BlockSpec((1, H), lambda i: (0, 0)),
        pl.BlockSpec((H, 32), lambda i: (0, 0)),
        pl.BlockSpec((1, 32), lambda i: (0, 0)),
        pl.BlockSpec((32, 1), lambda i: (0, 0)),
        pl.BlockSpec((1, 1), lambda i: (0, 0)),
    ],
    out_specs=pl.BlockSpec((_BE, 1), lambda i: (i, 0)),
    out_shape=jax.ShapeDtypeStruct((NE, 1), _F32),
)


def kernel(x, edge_index, edge_attr, Wl0, Wr0, b0, Wl1, Wr1, b1,
           Wl2, Wr2, b2, PW1, Pb1, PW2, Pb2, PW3, Pb3):
    src2 = edge_index[0].reshape(NCHUNK, CHE)
    dst2 = edge_index[1].reshape(NCHUNK, CHE)
    z2d = jnp.zeros((NN, W128), _F32)

    def wpad(Wl):
        return jnp.pad(Wl, ((0, 0), (0, W128 - H)))

    e64 = jnp.zeros((1, W128), _F32).at[0, H].set(1.0)
    WcP = jnp.concatenate([PW1[:H], PW1[H:2 * H]], axis=1)

    tab0, sf0 = _mm0(x, wpad(Wl0), e64, Wr0, b0.reshape(1, H))
    p1 = _sc_agg(tab0, src2, dst2, z2d)
    tab1, sf1, inv = _combine_first(p1, sf0, wpad(Wl1), Wr1, b1.reshape(1, H))
    p2 = _sc_agg(tab1, src2, dst2, z2d)
    tab2, sf2 = _combine_mid(p2, inv, sf1, wpad(Wl2), Wr2, b2.reshape(1, H))
    p3 = _sc_agg(tab2, src2, dst2, z2d)
    huv = _combine_last(p3, inv, sf2, WcP)
    gs, gd = _sc_gather(huv, src2, dst2)
    return _edge_mlp(gs, gd, edge_attr, PW1[2 * H:], Pb1.reshape(1, H),
                     PW2, Pb2.reshape(1, 32), PW3, Pb3.reshape(1, 1))
